# Initial kernel scaffold; baseline (speedup 1.0000x reference)
import jax
import jax.numpy as jnp
from jax.experimental import pallas as pl


def kernel(x, hex_idx):
    raise NotImplementedError("write your pallas kernel here")


# speedup vs baseline: 4.2687x; 4.2687x over previous
import functools

import jax
import jax.numpy as jnp
from jax import lax
from jax.experimental import pallas as pl
from jax.experimental.pallas import tpu as pltpu
from jax.experimental.pallas import tpu_sc as plsc

NC = 2
NS = 16
NW = NC * NS
K = 7
LANES = 16


@functools.lru_cache(maxsize=None)
def _build(n_verts: int, feat: int, L: int):
    G = 8
    bpw = ((L + NW * G - 1) // (NW * G)) * G
    L_pad = bpw * NW
    n_chunks = bpw // G
    mesh = plsc.VectorSubcoreMesh(
        core_axis_name="c", subcore_axis_name="s",
        num_cores=NC, num_subcores=NS)

    def body(x_hbm, idx_hbm, out_hbm, idx_v, rows_v, out_v, sem):
        wid = lax.axis_index("s") * NC + lax.axis_index("c")
        base = wid * bpw
        pltpu.sync_copy(idx_hbm.at[pl.ds(base * K, bpw * K)], idx_v)

        def chunk(c, carry):
            off = c * (G * K)
            pltpu.async_copy(
                x_hbm.at[idx_v.at[pl.ds(off, G * K)]], rows_v, sem
            ).wait()
            for g in range(G):
                for d in range(feat // LANES):
                    sl = pl.ds(d * LANES, LANES)
                    acc = rows_v[g * K, sl]
                    for j in range(1, K):
                        acc = jnp.maximum(acc, rows_v[g * K + j, sl])
                    out_v[g, sl] = acc
            pltpu.sync_copy(out_v, out_hbm.at[pl.ds(base + c * G, G)])
            return carry

        lax.fori_loop(0, n_chunks, chunk, 0)

    kern = pl.kernel(
        body,
        out_type=jax.ShapeDtypeStruct((L_pad, feat), jnp.float32),
        mesh=mesh,
        scratch_types=[
            pltpu.VMEM((bpw * K,), jnp.int32),
            pltpu.VMEM((G * K, feat), jnp.float32),
            pltpu.VMEM((G, feat), jnp.float32),
            pltpu.SemaphoreType.DMA,
        ],
    )
    return kern, L_pad


def kernel(x, hex_idx):
    n = hex_idx.shape[0]
    feat = x.shape[-1]
    x2 = x.reshape(n, -1)
    L = (n + 6) // 4
    kern, L_pad = _build(n, feat, L)
    idx = hex_idx[:L].astype(jnp.int32)
    idx = jnp.pad(idx, ((0, L_pad - L), (0, 0)))
    out = kern(x2, idx.reshape(-1))
    return out[:L]


# speedup vs baseline: 5.5963x; 1.3110x over previous
import functools

import jax
import jax.numpy as jnp
from jax import lax
from jax.experimental import pallas as pl
from jax.experimental.pallas import tpu as pltpu
from jax.experimental.pallas import tpu_sc as plsc

NC = 2
NS = 16
NW = NC * NS
K = 7
LANES = 16
G = 16
NBUF = 3


@functools.lru_cache(maxsize=None)
def _build(n_verts: int, feat: int, L: int):
    bpw = ((L + NW * G * NBUF - 1) // (NW * G * NBUF)) * (G * NBUF)
    L_pad = bpw * NW
    n_chunks = bpw // G
    n_outer = n_chunks // NBUF - 1
    mesh = plsc.VectorSubcoreMesh(
        core_axis_name="c", subcore_axis_name="s",
        num_cores=NC, num_subcores=NS)

    def body(x_hbm, idx_hbm, out_hbm,
             idx_v, r0, r1, r2, out_v, s0, s1, s2):
        rows = [r0, r1, r2]
        sems = [s0, s1, s2]
        wid = lax.axis_index("s") * NC + lax.axis_index("c")
        base = wid * bpw
        pltpu.sync_copy(idx_hbm.at[pl.ds(base * K, bpw * K)], idx_v)

        def gather_start(c, b):
            pltpu.async_copy(
                x_hbm.at[idx_v.at[pl.ds(c * (G * K), G * K)]],
                rows[b], sems[b])

        def gather_wait(b):
            pltpu.make_async_copy(
                x_hbm.at[idx_v.at[pl.ds(0, G * K)]],
                rows[b], sems[b]).wait()

        def compute_out(c, b):
            rv = rows[b]

            def row(g, carry):
                for d in range(feat // LANES):
                    sl = pl.ds(d * LANES, LANES)
                    acc = rv[g * K, sl]
                    for j in range(1, K):
                        acc = jnp.maximum(acc, rv[g * K + j, sl])
                    out_v[g, sl] = acc
                return carry

            lax.fori_loop(0, G, row, 0)
            pltpu.sync_copy(out_v, out_hbm.at[pl.ds(base + c * G, G)])

        for b in range(NBUF):
            gather_start(b, b)

        def outer(o, carry):
            for b in range(NBUF):
                c = o * NBUF + b
                gather_wait(b)
                compute_out(c, b)
                gather_start(c + NBUF, b)
            return carry

        lax.fori_loop(0, n_outer, outer, 0)

        for b in range(NBUF):
            c = n_outer * NBUF + b
            gather_wait(b)
            compute_out(c, b)

    kern = pl.kernel(
        body,
        out_type=jax.ShapeDtypeStruct((L_pad, feat), jnp.float32),
        mesh=mesh,
        scratch_types=[
            pltpu.VMEM((bpw * K,), jnp.int32),
            pltpu.VMEM((G * K, feat), jnp.float32),
            pltpu.VMEM((G * K, feat), jnp.float32),
            pltpu.VMEM((G * K, feat), jnp.float32),
            pltpu.VMEM((G, feat), jnp.float32),
            pltpu.SemaphoreType.DMA,
            pltpu.SemaphoreType.DMA,
            pltpu.SemaphoreType.DMA,
        ],
    )
    return kern, L_pad


def kernel(x, hex_idx):
    n = hex_idx.shape[0]
    feat = x.shape[-1]
    x2 = x.reshape(n, -1)
    L = (n + 6) // 4
    kern, L_pad = _build(n, feat, L)
    idx = hex_idx[:L].astype(jnp.int32)
    idx = jnp.pad(idx, ((0, L_pad - L), (0, 0)))
    out = kern(x2, idx.reshape(-1))
    return out[:L]
